# baseline (speedup 1.0000x reference)
import functools

import jax
import jax.numpy as jnp
import numpy as np
from jax import lax
from jax.experimental import pallas as pl
from jax.experimental.pallas import tpu as pltpu
from jax.experimental.pallas import tpu_sc as plsc

_N, _E, _D = 10000, 2000, 128
_TA = 400
_NTA = _N // _TA
_TB = 2000
_NTB = _N // _TB
_BF = jnp.bfloat16
_F32 = jnp.float32

_SC_NC, _SC_NS = 2, 16
_NW = _SC_NC * _SC_NS
_EPAD = 2048
_BPW = _EPAD // _NW


def _sc_root_gather(table, idx):
    mesh = plsc.VectorSubcoreMesh(core_axis_name="c", subcore_axis_name="s")

    @functools.partial(
        pl.kernel,
        mesh=mesh,
        out_type=jax.ShapeDtypeStruct((_EPAD, _D), jnp.float32),
        scratch_types=[
            pltpu.VMEM((_BPW,), jnp.int32),
            pltpu.VMEM((_BPW, _D), jnp.float32),
            pltpu.SemaphoreType.DMA,
        ],
    )
    def k(table_hbm, idx_hbm, out_hbm, idx_v, rows_v, sem):
        wid = lax.axis_index("s") * _SC_NC + lax.axis_index("c")
        base = wid * _BPW
        pltpu.sync_copy(idx_hbm.at[pl.ds(base, _BPW)], idx_v)
        pltpu.async_copy(table_hbm.at[idx_v], rows_v, sem).wait()
        pltpu.sync_copy(rows_v, out_hbm.at[pl.ds(base, _BPW)])

    return k(table, idx)


def _nn(a, b):
    return lax.dot_general(a, b, (((1,), (0,)), ((), ())),
                           preferred_element_type=_F32)


def _tn(a, b):
    return lax.dot_general(a, b, (((0,), (0,)), ((), ())),
                           preferred_element_type=_F32)


def _nt(a, b):
    return lax.dot_general(a, b, (((1,), (1,)), ((), ())),
                           preferred_element_type=_F32)


def _body_a(adj_ref, x_ref, root_ref, W2_ref, W3_ref, bns_ref, bnb_ref,
            pm_ref, eout_ref, erbf_ref,
            aTx_s, deg_s, e4a_s, eaccT_s, csum_s):
    i = pl.program_id(0)
    ones_c = jnp.ones((_TA, 1), _BF)

    @pl.when(i < _NTA)
    def _phase0():
        adj_b = adj_ref[...].astype(_BF)
        x_b = x_ref[...].astype(_BF)

        @pl.when(i == 0)
        def _init0():
            aTx_s[...] = jnp.zeros_like(aTx_s)
            deg_s[...] = jnp.zeros_like(deg_s)

        deg_s[...] += _tn(ones_c, adj_b)
        aTx_s[...] += _tn(adj_b, x_b)

    @pl.when(i >= _NTA)
    def _phase1():
        @pl.when(i == _NTA)
        def _init1():
            degc = jnp.transpose(deg_s[...])
            edge0 = aTx_s[...] / (degc + 1e-10) + root_ref[...]
            e4a_s[...] = _nn(edge0, W3_ref[...])
            eaccT_s[...] = jnp.zeros_like(eaccT_s)
            csum_s[...] = jnp.zeros_like(csum_s)

        x_t = x_ref[...]
        x_b = x_t.astype(_BF)
        x4a = _nn(x_t, W2_ref[...])
        att = _nt(x4a, e4a_s[...])
        pm = (jnp.exp(att) * adj_ref[...]).astype(_BF)
        pm_ref[...] = pm
        csum_s[...] += _tn(ones_c, pm)
        eaccT_s[...] += _tn(x_b, pm)

        @pl.when(i == 2 * _NTA - 1)
        def _fin1():
            er = jnp.transpose(eaccT_s[...] / csum_s[...])
            erbf_ref[...] = er.astype(_BF)
            e_elu = jnp.where(er > 0, er, jnp.exp(er) - 1.0)
            eout_ref[...] = e_elu * bns_ref[...] + bnb_ref[...]


def _body_b(pm_ref, x_ref, erbf_ref, bns_ref, bnb_ref,
            fw1_ref, fb1_ref, fw2_ref, fb2_ref, xout_ref):
    pm = pm_ref[...]
    x_t = x_ref[...]
    rsum = _nn(pm, jnp.ones((_E, 1), _BF))
    node = _nn(pm, erbf_ref[...]) / rsum
    node = jnp.where(node > 0, node, jnp.exp(node) - 1.0)
    node = node * bns_ref[...] + bnb_ref[...]
    h0 = jnp.tanh(jnp.dot(x_t, fw1_ref[...]) + fb1_ref[...])
    s0 = jnp.sum(h0 * fw2_ref[...], axis=1, keepdims=True) + fb2_ref[...]
    h1 = jnp.tanh(jnp.dot(node, fw1_ref[...]) + fb1_ref[...])
    s1 = jnp.sum(h1 * fw2_ref[...], axis=1, keepdims=True) + fb2_ref[...]
    mx = jnp.maximum(s0, s1)
    e0 = jnp.exp(s0 - mx)
    e1 = jnp.exp(s1 - mx)
    xout_ref[...] = (e0 * x_t + e1 * node) / (e0 + e1)


_A_IN_SPECS = [
    pl.BlockSpec((_TA, _E), lambda i: (i % _NTA, 0)),
    pl.BlockSpec((_TA, _D), lambda i: (i % _NTA, 0)),
    pl.BlockSpec((_E, _D), lambda i: (0, 0)),
    pl.BlockSpec((_D, _D), lambda i: (0, 0)),
    pl.BlockSpec((_D, _D), lambda i: (0, 0)),
    pl.BlockSpec((1, _D), lambda i: (0, 0)),
    pl.BlockSpec((1, _D), lambda i: (0, 0)),
]

_A_OUT_SPECS = [
    pl.BlockSpec((_TA, _E), lambda i: (jnp.maximum(i - _NTA, 0), 0)),
    pl.BlockSpec((_E, _D), lambda i: (0, 0)),
    pl.BlockSpec((_E, _D), lambda i: (0, 0)),
]

_A_OUT_SHAPE = [
    jax.ShapeDtypeStruct((_N, _E), _BF),
    jax.ShapeDtypeStruct((_E, _D), jnp.float32),
    jax.ShapeDtypeStruct((_E, _D), _BF),
]

_A_SCRATCH = [
    pltpu.VMEM((_E, _D), jnp.float32),
    pltpu.VMEM((1, _E), jnp.float32),
    pltpu.VMEM((_E, _D), jnp.float32),
    pltpu.VMEM((_D, _E), jnp.float32),
    pltpu.VMEM((1, _E), jnp.float32),
]

_B_IN_SPECS = [
    pl.BlockSpec((_TB, _E), lambda i: (i, 0)),
    pl.BlockSpec((_TB, _D), lambda i: (i, 0)),
    pl.BlockSpec((_E, _D), lambda i: (0, 0)),
    pl.BlockSpec((1, _D), lambda i: (0, 0)),
    pl.BlockSpec((1, _D), lambda i: (0, 0)),
    pl.BlockSpec((_D, _D), lambda i: (0, 0)),
    pl.BlockSpec((1, _D), lambda i: (0, 0)),
    pl.BlockSpec((1, _D), lambda i: (0, 0)),
    pl.BlockSpec((1, 1), lambda i: (0, 0)),
]


def kernel(x, adj, root_index, W2, W3, bn_gamma, bn_beta, bn_mean, bn_var,
           fw1, fb1, fw2, fb2):
    idx = jnp.concatenate([root_index.astype(jnp.int32),
                           jnp.zeros((_EPAD - _E,), jnp.int32)])
    root_emb = _sc_root_gather(x, idx)[:_E]

    bn_scale = (bn_gamma * lax.rsqrt(bn_var + 1e-5)).reshape(1, _D)
    bn_shift = (bn_beta - bn_mean * bn_scale[0]).reshape(1, _D)

    pm, edge_out, er_bf = pl.pallas_call(
        _body_a,
        grid=(2 * _NTA,),
        in_specs=_A_IN_SPECS,
        out_specs=_A_OUT_SPECS,
        out_shape=_A_OUT_SHAPE,
        scratch_shapes=_A_SCRATCH,
        compiler_params=pltpu.CompilerParams(
            dimension_semantics=("arbitrary",),
            vmem_limit_bytes=64 * 1024 * 1024,
        ),
    )(adj, x, root_emb, W2, W3, bn_scale, bn_shift)

    x_out = pl.pallas_call(
        _body_b,
        grid=(_NTB,),
        in_specs=_B_IN_SPECS,
        out_specs=pl.BlockSpec((_TB, _D), lambda i: (i, 0)),
        out_shape=jax.ShapeDtypeStruct((_N, _D), jnp.float32),
        compiler_params=pltpu.CompilerParams(
            dimension_semantics=("arbitrary",),
            vmem_limit_bytes=64 * 1024 * 1024,
        ),
    )(pm, x, er_bf, bn_scale, bn_shift, fw1, fb1.reshape(1, _D),
      fw2.reshape(1, _D), fb2.reshape(1, 1))
    return x_out, edge_out


# speedup vs baseline: 1.7867x; 1.7867x over previous
import functools

import jax
import jax.numpy as jnp
import numpy as np
from jax import lax
from jax.experimental import pallas as pl
from jax.experimental.pallas import tpu as pltpu
from jax.experimental.pallas import tpu_sc as plsc

_N, _E, _D = 10000, 2000, 128
_TE = 200
_NE = _E // _TE
_TN = 1000
_NN = _N // _TN
_BF = jnp.bfloat16
_F32 = jnp.float32

_SC_NC, _SC_NS = 2, 16
_NW = _SC_NC * _SC_NS
_EPAD = 2048
_BPW = _EPAD // _NW


def _sc_root_gather(table, idx):
    mesh = plsc.VectorSubcoreMesh(core_axis_name="c", subcore_axis_name="s")

    @functools.partial(
        pl.kernel,
        mesh=mesh,
        out_type=jax.ShapeDtypeStruct((_EPAD, _D), jnp.float32),
        scratch_types=[
            pltpu.VMEM((_BPW,), jnp.int32),
            pltpu.VMEM((_BPW, _D), jnp.float32),
            pltpu.SemaphoreType.DMA,
        ],
    )
    def k(table_hbm, idx_hbm, out_hbm, idx_v, rows_v, sem):
        wid = lax.axis_index("s") * _SC_NC + lax.axis_index("c")
        base = wid * _BPW
        pltpu.sync_copy(idx_hbm.at[pl.ds(base, _BPW)], idx_v)
        pltpu.async_copy(table_hbm.at[idx_v], rows_v, sem).wait()
        pltpu.sync_copy(rows_v, out_hbm.at[pl.ds(base, _BPW)])

    return k(table, idx)


def _nn(a, b):
    return lax.dot_general(a, b, (((1,), (0,)), ((), ())),
                           preferred_element_type=_F32)


def _tn(a, b):
    return lax.dot_general(a, b, (((0,), (0,)), ((), ())),
                           preferred_element_type=_F32)


def _nt(a, b):
    return lax.dot_general(a, b, (((1,), (1,)), ((), ())),
                           preferred_element_type=_F32)


def _tc_body(adjT_ref, xf_ref, xb_ref, root_ref, W2_ref, W3_ref,
             bns_ref, bnb_ref, fw1_ref, fb1_ref, fw2_ref, fb2_ref,
             eout_ref, xout_ref,
             xbf_s, x4a_s, nacc_s, rsum_s, rsumT_s):
    i = pl.program_id(0)

    @pl.when(i < _NE)
    def _main():
        @pl.when(i == 0)
        def _init():
            xf = xf_ref[...]
            xbf_s[...] = xf.astype(_BF)
            x4a_s[...] = _nn(xf, W2_ref[...])
            nacc_s[...] = jnp.zeros_like(nacc_s)
            rsum_s[...] = jnp.zeros_like(rsum_s)

        adjT = adjT_ref[...]
        adjT_b = adjT.astype(_BF)
        x_b = xbf_s[...]
        deg = _nn(adjT_b, jnp.ones((_N, 1), _BF))
        aTx = _nn(adjT_b, x_b)
        edge0 = aTx / (deg + 1e-10) + root_ref[...]
        e4a = _nn(edge0, W3_ref[...])
        attT = _nt(e4a, x4a_s[...])
        pm = (jnp.exp(attT) * adjT).astype(_BF)
        csum = _nn(pm, jnp.ones((_N, 1), _BF))
        er = _nn(pm, x_b) / csum
        e_elu = jnp.where(er > 0, er, jnp.exp(er) - 1.0)
        eout_ref[...] = e_elu * bns_ref[...] + bnb_ref[...]
        er_b = er.astype(_BF)
        nacc_s[...] += _tn(pm, er_b)
        rsum_s[...] += _tn(jnp.ones((_TE, 1), _BF), pm)

    @pl.when(i >= _NE)
    def _epi():
        s = i - _NE

        @pl.when(i == _NE)
        def _trans():
            rsumT_s[...] = jnp.transpose(rsum_s[...])

        x_t = xb_ref[...]
        rs = rsumT_s[pl.ds(s * _TN, _TN), :]
        node = nacc_s[pl.ds(s * _TN, _TN), :] / rs
        node = jnp.where(node > 0, node, jnp.exp(node) - 1.0)
        node = node * bns_ref[...] + bnb_ref[...]
        h0 = jnp.tanh(jnp.dot(x_t, fw1_ref[...]) + fb1_ref[...])
        s0 = jnp.sum(h0 * fw2_ref[...], axis=1, keepdims=True) + fb2_ref[...]
        h1 = jnp.tanh(jnp.dot(node, fw1_ref[...]) + fb1_ref[...])
        s1 = jnp.sum(h1 * fw2_ref[...], axis=1, keepdims=True) + fb2_ref[...]
        mx = jnp.maximum(s0, s1)
        e0 = jnp.exp(s0 - mx)
        e1 = jnp.exp(s1 - mx)
        xout_ref[...] = (e0 * x_t + e1 * node) / (e0 + e1)


_IN_SPECS = [
    pl.BlockSpec((_TE, _N), lambda i: (jnp.minimum(i, _NE - 1), 0)),
    pl.BlockSpec((_N, _D), lambda i: (0, 0)),
    pl.BlockSpec((_TN, _D),
                 lambda i: (jnp.maximum(i - _NE, 0), 0)),
    pl.BlockSpec((_TE, _D), lambda i: (jnp.minimum(i, _NE - 1), 0)),
    pl.BlockSpec((_D, _D), lambda i: (0, 0)),
    pl.BlockSpec((_D, _D), lambda i: (0, 0)),
    pl.BlockSpec((1, _D), lambda i: (0, 0)),
    pl.BlockSpec((1, _D), lambda i: (0, 0)),
    pl.BlockSpec((_D, _D), lambda i: (0, 0)),
    pl.BlockSpec((1, _D), lambda i: (0, 0)),
    pl.BlockSpec((1, _D), lambda i: (0, 0)),
    pl.BlockSpec((1, 1), lambda i: (0, 0)),
]

_OUT_SPECS = [
    pl.BlockSpec((_TE, _D), lambda i: (jnp.minimum(i, _NE - 1), 0)),
    pl.BlockSpec((_TN, _D), lambda i: (jnp.maximum(i - _NE, 0), 0)),
]

_OUT_SHAPE = [
    jax.ShapeDtypeStruct((_E, _D), jnp.float32),
    jax.ShapeDtypeStruct((_N, _D), jnp.float32),
]

_SCRATCH = [
    pltpu.VMEM((_N, _D), _BF),
    pltpu.VMEM((_N, _D), jnp.float32),
    pltpu.VMEM((_N, _D), jnp.float32),
    pltpu.VMEM((1, _N), jnp.float32),
    pltpu.VMEM((_N, 1), jnp.float32),
]


def kernel(x, adj, root_index, W2, W3, bn_gamma, bn_beta, bn_mean, bn_var,
           fw1, fb1, fw2, fb2):
    idx = jnp.concatenate([root_index.astype(jnp.int32),
                           jnp.zeros((_EPAD - _E,), jnp.int32)])
    root_emb = _sc_root_gather(x, idx)

    bn_scale = (bn_gamma * lax.rsqrt(bn_var + 1e-5)).reshape(1, _D)
    bn_shift = (bn_beta - bn_mean * bn_scale[0]).reshape(1, _D)

    adjT = jnp.swapaxes(adj, 0, 1)

    edge_out, x_out = pl.pallas_call(
        _tc_body,
        grid=(_NE + _NN,),
        in_specs=_IN_SPECS,
        out_specs=_OUT_SPECS,
        out_shape=_OUT_SHAPE,
        scratch_shapes=_SCRATCH,
        compiler_params=pltpu.CompilerParams(
            dimension_semantics=("arbitrary",),
            vmem_limit_bytes=64 * 1024 * 1024,
        ),
    )(adjT, x, x, root_emb, W2, W3, bn_scale, bn_shift, fw1,
      fb1.reshape(1, _D), fw2.reshape(1, _D), fb2.reshape(1, 1))
    return x_out, edge_out
